# baseline (speedup 1.0000x reference)
import functools

import jax
import jax.numpy as jnp
from jax import lax
from jax.experimental import pallas as pl
from jax.experimental.pallas import tpu as pltpu
from jax.experimental.pallas import tpu_sc as plsc

BATCH = 16384
EMBED_DIM = 128
ID_DIM = 64
NUM_FEATS = 20
HIDDEN = 128

_NC = 2
_NS = 16
_NW = _NC * _NS
_B_PER_W = BATCH // _NW
_IDX_CHUNK = 128


def _gather_sc(ids, table):
    mesh = plsc.VectorSubcoreMesh(core_axis_name="c", subcore_axis_name="s")

    @functools.partial(
        pl.kernel,
        mesh=mesh,
        compiler_params=pltpu.CompilerParams(use_tc_tiling_on_sc=False),
        out_type=jax.ShapeDtypeStruct((BATCH, ID_DIM), jnp.float32),
        scratch_types=[
            pltpu.VMEM((_B_PER_W,), jnp.int32),
            pltpu.VMEM((_B_PER_W, ID_DIM), jnp.float32),
            pltpu.SemaphoreType.DMA,
        ],
    )
    def gather_kernel(idx_hbm, table_hbm, out_hbm, idx_v, rows_v, sem):
        wid = lax.axis_index("s") * _NC + lax.axis_index("c")
        base = wid * _B_PER_W
        pltpu.sync_copy(idx_hbm.at[pl.ds(base, _B_PER_W)], idx_v)
        copies = []
        for j in range(_B_PER_W // _IDX_CHUNK):
            sl = pl.ds(j * _IDX_CHUNK, _IDX_CHUNK)
            copies.append(
                pltpu.async_copy(table_hbm.at[idx_v.at[sl]], rows_v.at[sl], sem)
            )
        for c in copies:
            c.wait()
        pltpu.sync_copy(rows_v, out_hbm.at[pl.ds(base, _B_PER_W)])

    return gather_kernel(ids, table)


def _mlp_body(emb_ref, feat_ref, w1a_ref, w1b_ref, b1_ref, w2_ref, b2_ref, out_ref):
    h = jnp.dot(emb_ref[...], w1a_ref[...], preferred_element_type=jnp.float32)
    h += jnp.dot(feat_ref[...], w1b_ref[...], preferred_element_type=jnp.float32)
    h = jnp.maximum(h + b1_ref[...], 0.0)
    out = jnp.dot(h, w2_ref[...], preferred_element_type=jnp.float32) + b2_ref[...]
    norm = jnp.sqrt(jnp.sum(out * out, axis=1, keepdims=True))
    out_ref[...] = out / jnp.maximum(norm, 1e-12)


_BB = 2048


def _mlp_tc(emb, feats, w1a, w1b, b1, w2, b2):
    grid = (BATCH // _BB,)
    return pl.pallas_call(
        _mlp_body,
        grid=grid,
        in_specs=[
            pl.BlockSpec((_BB, ID_DIM), lambda i: (i, 0)),
            pl.BlockSpec((_BB, NUM_FEATS), lambda i: (i, 0)),
            pl.BlockSpec((ID_DIM, HIDDEN), lambda i: (0, 0)),
            pl.BlockSpec((NUM_FEATS, HIDDEN), lambda i: (0, 0)),
            pl.BlockSpec((1, HIDDEN), lambda i: (0, 0)),
            pl.BlockSpec((HIDDEN, EMBED_DIM), lambda i: (0, 0)),
            pl.BlockSpec((1, EMBED_DIM), lambda i: (0, 0)),
        ],
        out_specs=pl.BlockSpec((_BB, EMBED_DIM), lambda i: (i, 0)),
        out_shape=jax.ShapeDtypeStruct((BATCH, EMBED_DIM), jnp.float32),
    )(emb, feats, w1a, w1b, b1, w2, b2)


def kernel(customer_ids, numerical_features, emb_table, W1, b1, W2, b2):
    ids = customer_ids.astype(jnp.int32)
    emb = _gather_sc(ids, emb_table)
    w1a = W1[:ID_DIM]
    w1b = W1[ID_DIM:]
    return _mlp_tc(
        emb,
        numerical_features,
        w1a,
        w1b,
        b1.reshape(1, HIDDEN),
        W2,
        b2.reshape(1, EMBED_DIM),
    )
